# baseline (speedup 1.0000x reference)
import functools

import jax
import jax.numpy as jnp
import numpy as np
from jax import lax
from jax.experimental import pallas as pl
from jax.experimental.pallas import tpu as pltpu
from jax.experimental.pallas import tpu_sc as plsc

N_NODES = 10000
N_EDGES = 320000
D = 128
D_EDGE = 16
HID = 14
HID_PAD = 16

NC = 2
NS = 16
NW = NC * NS
EPW = N_EDGES // NW
CHUNK = 80
NCHUNK = EPW // CHUNK
STRIPE = 624
STRIPE_REM = N_NODES - NS * STRIPE
Z_ROWS = 16

_SC_MESH = dict(core_axis_name="c", subcore_axis_name="s")


def _sc_gather_xixj(atom, src3, dst3):
    nw, nchunk, chunk = src3.shape
    epw = nchunk * chunk
    n_edges = nw * epw

    @functools.partial(
        pl.kernel,
        out_type=[jax.ShapeDtypeStruct((n_edges, D), jnp.float32),
                  jax.ShapeDtypeStruct((n_edges, D), jnp.float32)],
        mesh=plsc.VectorSubcoreMesh(**_SC_MESH),
        scratch_types=[
            pltpu.VMEM((nchunk, chunk), jnp.int32),
            pltpu.VMEM((nchunk, chunk), jnp.int32),
            pltpu.VMEM((chunk, D), jnp.float32),
            pltpu.VMEM((chunk, D), jnp.float32),
            pltpu.VMEM((chunk, D), jnp.float32),
            pltpu.VMEM((chunk, D), jnp.float32),
            pltpu.SemaphoreType.DMA,
            pltpu.SemaphoreType.DMA,
        ],
    )
    def k(atom_hbm, src_hbm, dst_hbm, xj_hbm, xi_hbm,
          idx_s, idx_d, s_a, d_a, s_b, d_b, sem_a, sem_b):
        wid = lax.axis_index("s") * NC + lax.axis_index("c")
        base = wid * epw
        pltpu.sync_copy(src_hbm.at[wid], idx_s)
        pltpu.sync_copy(dst_hbm.at[wid], idx_d)

        def issue(j, bs, bd, sem):
            pltpu.async_copy(atom_hbm.at[idx_s.at[j]], bs, sem)
            pltpu.async_copy(atom_hbm.at[idx_d.at[j]], bd, sem)

        def drain(bs, bd, sem):
            pltpu.make_async_copy(atom_hbm.at[pl.ds(0, chunk)], bs, sem).wait()
            pltpu.make_async_copy(atom_hbm.at[pl.ds(0, chunk)], bd, sem).wait()

        def wb(j, bs, bd):
            off = base + j * chunk
            pltpu.sync_copy(bs, xj_hbm.at[pl.ds(off, chunk)])
            pltpu.sync_copy(bd, xi_hbm.at[pl.ds(off, chunk)])

        issue(0, s_a, d_a, sem_a)

        def body(t, _):
            j = 2 * t
            issue(j + 1, s_b, d_b, sem_b)
            drain(s_a, d_a, sem_a)
            wb(j, s_a, d_a)

            @pl.when(j + 2 < nchunk)
            def _():
                issue(j + 2, s_a, d_a, sem_a)

            drain(s_b, d_b, sem_b)
            wb(j + 1, s_b, d_b)
            return 0

        lax.fori_loop(0, nchunk // 2, body, 0)
        drain(s_a, d_a, sem_a)
        wb(nchunk - 1, s_a, d_a)

    return k(atom, src3, dst3)


_MSG_BLK = 2000


def _tc_message(xi, xj, ef, wfi, wfj, wfe, bfv, wsi, wsj, wse, bsv):
    n_edges = xi.shape[0]
    def body(xi_ref, xj_ref, ef_ref, wfi_ref, wfj_ref, wfe_ref, bf_ref,
             wsi_ref, wsj_ref, wse_ref, bs_ref, out_ref):
        xi_b = xi_ref[...]
        xj_b = xj_ref[...]
        ef_b = ef_ref[...]
        dot = functools.partial(jnp.dot, preferred_element_type=jnp.float32)
        pf = (dot(xi_b, wfi_ref[...]) + dot(xj_b, wfj_ref[...])
              + dot(ef_b, wfe_ref[...]) + bf_ref[...])
        ps = (dot(xi_b, wsi_ref[...]) + dot(xj_b, wsj_ref[...])
              + dot(ef_b, wse_ref[...]) + bs_ref[...])
        sp = jnp.maximum(ps, 0.0) + jnp.log1p(jnp.exp(-jnp.abs(ps)))
        out_ref[...] = jax.nn.sigmoid(pf) * sp

    full = lambda shape: pl.BlockSpec(shape, lambda i: (0, 0))
    return pl.pallas_call(
        body,
        grid=(n_edges // _MSG_BLK,),
        in_specs=[
            pl.BlockSpec((_MSG_BLK, D), lambda i: (i, 0)),
            pl.BlockSpec((_MSG_BLK, D), lambda i: (i, 0)),
            pl.BlockSpec((_MSG_BLK, D_EDGE), lambda i: (i, 0)),
            full((D, D)), full((D, D)), full((D_EDGE, D)), full((1, D)),
            full((D, D)), full((D, D)), full((D_EDGE, D)), full((1, D)),
        ],
        out_specs=pl.BlockSpec((_MSG_BLK, D), lambda i: (i, 0)),
        out_shape=jax.ShapeDtypeStruct((n_edges, D), jnp.float32),
        compiler_params=pltpu.CompilerParams(
            dimension_semantics=("arbitrary",)),
    )(xi, xj, ef, wfi, wfj, wfe, bfv, wsi, wsj, wse, bsv)


def _sc_scatter_add(msg, dst3):
    nw, nchunk, chunk = dst3.shape
    epw = nchunk * chunk

    @functools.partial(
        pl.kernel,
        out_type=jax.ShapeDtypeStruct((NC, N_NODES, D), jnp.float32),
        mesh=plsc.VectorSubcoreMesh(**_SC_MESH),
        scratch_types=[
            pltpu.VMEM((chunk, D), jnp.float32),
            pltpu.VMEM((chunk, D), jnp.float32),
            pltpu.VMEM((nchunk, chunk), jnp.int32),
            pltpu.VMEM((Z_ROWS, D), jnp.float32),
            pltpu.VMEM_SHARED((N_NODES, D), jnp.float32),
            pltpu.SemaphoreType.DMA,
            pltpu.SemaphoreType.DMA,
        ],
    )
    def k(msg_hbm, dst_hbm, out_hbm, m_a, m_b, idxbuf, zbuf, agg,
          sem_a, sem_b):
        c = lax.axis_index("c")
        s = lax.axis_index("s")
        wid = s * NC + c

        zero = jnp.zeros((16,), jnp.float32)

        def zrow(r, _):
            for cc in range(D // 16):
                zbuf[r, pl.ds(cc * 16, 16)] = zero
            return 0

        lax.fori_loop(0, Z_ROWS, zrow, 0)
        for t in range(STRIPE // Z_ROWS):
            pltpu.sync_copy(zbuf, agg.at[pl.ds(s * STRIPE + t * Z_ROWS, Z_ROWS)])

        @pl.when(s == NS - 1)
        def _zero_tail():
            pltpu.sync_copy(zbuf.at[pl.ds(0, STRIPE_REM)],
                            agg.at[pl.ds(NS * STRIPE, STRIPE_REM)])

        plsc.subcore_barrier()

        pltpu.sync_copy(dst_hbm.at[wid], idxbuf)
        base = wid * epw

        def issue(j, buf, sem):
            pltpu.async_copy(msg_hbm.at[pl.ds(base + j * chunk, chunk)],
                             buf, sem)

        def drain(buf, sem):
            pltpu.make_async_copy(msg_hbm.at[pl.ds(0, chunk)], buf, sem).wait()

        def scat(j, buf):
            pltpu.sync_copy(buf, agg.at[idxbuf.at[j]], add=True)

        issue(0, m_a, sem_a)

        def body(t, _):
            j = 2 * t
            issue(j + 1, m_b, sem_b)
            drain(m_a, sem_a)
            scat(j, m_a)

            @pl.when(j + 2 < nchunk)
            def _():
                issue(j + 2, m_a, sem_a)

            drain(m_b, sem_b)
            scat(j + 1, m_b)
            return 0

        lax.fori_loop(0, nchunk // 2, body, 0)
        drain(m_a, sem_a)
        scat(nchunk - 1, m_a)
        plsc.subcore_barrier()

        pltpu.sync_copy(agg.at[pl.ds(s * STRIPE, STRIPE)],
                        out_hbm.at[c, pl.ds(s * STRIPE, STRIPE)])

        @pl.when(s == NS - 1)
        def _flush_tail():
            pltpu.sync_copy(agg.at[pl.ds(NS * STRIPE, STRIPE_REM)],
                            out_hbm.at[c, pl.ds(NS * STRIPE, STRIPE_REM)])

    return k(msg, dst3)


def _tc_node_update(pa, pb, atom, w1ab):
    def body(pa_ref, pb_ref, atom_ref, w1ab_ref, out_ref, q_ref):
        a = pa_ref[...]
        b = pb_ref[...]
        ao = a[0] + a[1] + b[0] + b[1] + atom_ref[...]
        out_ref[...] = ao
        q_ref[...] = ao @ w1ab_ref[...]

    return pl.pallas_call(
        body,
        out_shape=[jax.ShapeDtypeStruct((N_NODES, D), jnp.float32),
                   jax.ShapeDtypeStruct((N_NODES, D), jnp.float32)],
    )(pa, pb, atom, w1ab)


QCH = 64
NQCH = N_EDGES // QCH
QROWS = QCH * HID_PAD // D
H_ROWS = N_EDGES * HID_PAD // D


def _sc_gather_h(qtab, src, dst):
    @functools.partial(
        pl.kernel,
        out_type=jax.ShapeDtypeStruct((H_ROWS, D), jnp.float32),
        mesh=plsc.VectorSubcoreMesh(**_SC_MESH),
        scratch_types=[
            pltpu.VMEM((QCH,), jnp.int32),
            pltpu.VMEM((QCH,), jnp.int32),
            pltpu.VMEM((QCH,), jnp.int32),
            pltpu.VMEM((QCH,), jnp.int32),
            pltpu.VMEM((QCH, D), jnp.float32),
            pltpu.VMEM((QCH, D), jnp.float32),
            pltpu.VMEM((QCH, D), jnp.float32),
            pltpu.VMEM((QCH, D), jnp.float32),
            pltpu.VMEM((QROWS, D), jnp.float32),
            pltpu.VMEM((QROWS, D), jnp.float32),
            pltpu.SemaphoreType.DMA,
            pltpu.SemaphoreType.DMA,
        ],
    )
    def k(q_hbm, src_hbm, dst_hbm, h_hbm,
          is_a, id_a, is_b, id_b, s_a, d_a, s_b, d_b, h_a, h_b,
          sem_a, sem_b):
        wid = lax.axis_index("s") * NC + lax.axis_index("c")

        def issue(g, isx, idx, bs, bd, sem):
            pltpu.sync_copy(src_hbm.at[pl.ds(g * QCH, QCH)], isx)
            pltpu.sync_copy(dst_hbm.at[pl.ds(g * QCH, QCH)], idx)
            pltpu.async_copy(q_hbm.at[isx], bs, sem)
            pltpu.async_copy(q_hbm.at[idx], bd, sem)

        def drain(bs, bd, sem):
            pltpu.make_async_copy(q_hbm.at[pl.ds(0, QCH)], bs, sem).wait()
            pltpu.make_async_copy(q_hbm.at[pl.ds(0, QCH)], bd, sem).wait()

        def addwb(g, bs, bd, hb):
            for e in range(QCH):
                hb[e // 8, pl.ds((e % 8) * HID_PAD, HID_PAD)] = (
                    bs[e, pl.ds(0, HID_PAD)] + bd[e, pl.ds(HID_PAD, HID_PAD)])
            pltpu.sync_copy(hb, h_hbm.at[pl.ds(g * QROWS, QROWS)])

        issue(wid, is_a, id_a, s_a, d_a, sem_a)

        def body(t, _):
            g0 = (2 * t) * NW + wid
            g1 = g0 + NW
            g2 = g1 + NW
            issue(g1, is_b, id_b, s_b, d_b, sem_b)
            drain(s_a, d_a, sem_a)
            addwb(g0, s_a, d_a, h_a)

            @pl.when(g2 < NQCH)
            def _():
                issue(g2, is_a, id_a, s_a, d_a, sem_a)

            drain(s_b, d_b, sem_b)
            addwb(g1, s_b, d_b, h_b)
            return 0

        lax.fori_loop(0, 78, body, 0)
        glast = 156 * NW + wid

        @pl.when(glast < NQCH)
        def _tail():
            drain(s_a, d_a, sem_a)
            addwb(glast, s_a, d_a, h_a)

    return k(qtab, src, dst)


_EDGE_BLK = 2560


def _tc_edge_mlp(h, ef, w1e, b1v, w2, b2v):
    def body(h_ref, ef_ref, w1e_ref, b1_ref, w2_ref, b2_ref, out_ref):
        hp = h_ref[...] + ef_ref[...] @ w1e_ref[...] + b1_ref[...]
        hh = hp * jax.nn.sigmoid(hp)
        o = hh @ w2_ref[...] + b2_ref[...]
        out_ref[...] = o * jax.nn.sigmoid(o)

    full = lambda shape: pl.BlockSpec(shape, lambda i: (0, 0))
    return pl.pallas_call(
        body,
        grid=(N_EDGES // _EDGE_BLK,),
        in_specs=[
            pl.BlockSpec((_EDGE_BLK, HID_PAD), lambda i: (i, 0)),
            pl.BlockSpec((_EDGE_BLK, D_EDGE), lambda i: (i, 0)),
            full((D_EDGE, HID_PAD)), full((1, HID_PAD)),
            full((HID_PAD, D)), full((1, D)),
        ],
        out_specs=pl.BlockSpec((_EDGE_BLK, D), lambda i: (i, 0)),
        out_shape=jax.ShapeDtypeStruct((N_EDGES, D), jnp.float32),
        compiler_params=pltpu.CompilerParams(
            dimension_semantics=("arbitrary",)),
    )(h, ef, w1e, b1v, w2, b2v)


def kernel(atom_fea, edge_idx, edge_fea, batch, distance, edge_vec,
           Wf, bf, Ws, bs, W1, b1, W2, b2):
    src = edge_idx[0].astype(jnp.int32)
    dst = edge_idx[1].astype(jnp.int32)

    EH = N_EDGES // 2
    CH_H = 40
    NCH_H = EH // NW // CH_H
    halves = []
    for p in range(2):
        s3 = lax.dynamic_slice_in_dim(src, p * EH, EH).reshape(NW, NCH_H, CH_H)
        d3 = lax.dynamic_slice_in_dim(dst, p * EH, EH).reshape(NW, NCH_H, CH_H)
        halves.append((s3, d3))

    wfi, wfj, wfe = Wf[:, :D].T, Wf[:, D:2 * D].T, Wf[:, 2 * D:].T
    wsi, wsj, wse = Ws[:, :D].T, Ws[:, D:2 * D].T, Ws[:, 2 * D:].T

    parts = []
    for p, (s3, d3) in enumerate(halves):
        xj, xi = _sc_gather_xixj(atom_fea, s3, d3)
        ef_h = lax.dynamic_slice_in_dim(edge_fea, p * EH, EH)
        msg = _tc_message(xi, xj, ef_h, wfi, wfj, wfe, bf.reshape(1, D),
                          wsi, wsj, wse, bs.reshape(1, D))
        parts.append(_sc_scatter_add(msg, d3))

    pad = jnp.zeros((D, HID_PAD - HID), jnp.float32)
    w1ab = jnp.concatenate(
        [W1[:, :D].T, pad, W1[:, D:2 * D].T, pad,
         jnp.zeros((D, D - 2 * HID_PAD), jnp.float32)], axis=1)
    atom_out, qtab = _tc_node_update(parts[0], parts[1], atom_fea, w1ab)

    h = _sc_gather_h(qtab, src, dst).reshape(N_EDGES, HID_PAD)

    epad = jnp.zeros((D_EDGE, HID_PAD - HID), jnp.float32)
    w1e = jnp.concatenate([W1[:, 2 * D:].T, epad], axis=1)
    b1v = jnp.concatenate([b1, jnp.zeros((HID_PAD - HID,), jnp.float32)])
    w2 = jnp.concatenate([W2.T, jnp.zeros((HID_PAD - HID, D), jnp.float32)],
                         axis=0)
    edge_out = _tc_edge_mlp(h, edge_fea, w1e, b1v.reshape(1, HID_PAD),
                            w2, b2.reshape(1, D))
    return atom_out, edge_out


# speedup vs baseline: 1.0915x; 1.0915x over previous
import functools

import jax
import jax.numpy as jnp
import numpy as np
from jax import lax
from jax.experimental import pallas as pl
from jax.experimental.pallas import tpu as pltpu
from jax.experimental.pallas import tpu_sc as plsc

N_NODES = 10000
N_EDGES = 320000
D = 128
D_EDGE = 16
HID = 14
HID_PAD = 16

NC = 2
NS = 16
NW = NC * NS
EPW = N_EDGES // NW
CHUNK = 80
NCHUNK = EPW // CHUNK
STRIPE = 624
STRIPE_REM = N_NODES - NS * STRIPE
Z_ROWS = 16

_SC_MESH = dict(core_axis_name="c", subcore_axis_name="s")


_NSET = 4


def _sc_gather_xixj(atom, src3, dst3):
    @functools.partial(
        pl.kernel,
        out_type=[jax.ShapeDtypeStruct((N_EDGES, D), jnp.float32),
                  jax.ShapeDtypeStruct((N_EDGES, D), jnp.float32)],
        mesh=plsc.VectorSubcoreMesh(**_SC_MESH),
        scratch_types=(
            [pltpu.VMEM((NCHUNK, CHUNK), jnp.int32)] * 2
            + [pltpu.VMEM((CHUNK, D), jnp.float32)] * (2 * _NSET)
            + [pltpu.SemaphoreType.DMA] * (2 * _NSET)
        ),
    )
    def k(atom_hbm, src_hbm, dst_hbm, xj_hbm, xi_hbm, idx_s, idx_d, *rest):
        bufs = rest[:2 * _NSET]
        gsems = rest[2 * _NSET:3 * _NSET]
        wsems = rest[3 * _NSET:]
        sets = [(bufs[2 * i], bufs[2 * i + 1], gsems[i], wsems[i])
                for i in range(_NSET)]
        wid = lax.axis_index("s") * NC + lax.axis_index("c")
        base = wid * EPW
        pltpu.sync_copy(src_hbm.at[wid], idx_s)
        pltpu.sync_copy(dst_hbm.at[wid], idx_d)

        def gi(j, S):
            bs, bd, gs, _ = S
            pltpu.async_copy(atom_hbm.at[idx_s.at[j]], bs, gs)
            pltpu.async_copy(atom_hbm.at[idx_d.at[j]], bd, gs)

        def gdrain(S):
            bs, bd, gs, _ = S
            pltpu.make_async_copy(atom_hbm.at[pl.ds(0, CHUNK)], bs, gs).wait()
            pltpu.make_async_copy(atom_hbm.at[pl.ds(0, CHUNK)], bd, gs).wait()

        def wbi(j, S):
            bs, bd, _, ws = S
            off = base + j * CHUNK
            pltpu.async_copy(bs, xj_hbm.at[pl.ds(off, CHUNK)], ws)
            pltpu.async_copy(bd, xi_hbm.at[pl.ds(off, CHUNK)], ws)

        def wdrain(S):
            bs, bd, _, ws = S
            pltpu.make_async_copy(bs, xj_hbm.at[pl.ds(0, CHUNK)], ws).wait()
            pltpu.make_async_copy(bd, xi_hbm.at[pl.ds(0, CHUNK)], ws).wait()

        for kk in range(_NSET):
            gi(kk, sets[kk])

        nbody = (NCHUNK - 1) // _NSET - 1

        def body(t, _):
            j = _NSET * t
            for kk in range(_NSET):
                gdrain(sets[kk])
                wbi(j + kk, sets[kk])
            for kk in range(_NSET):
                wdrain(sets[kk])
                gi(j + _NSET + kk, sets[kk])
            return 0

        lax.fori_loop(0, nbody, body, 0)
        jlast = _NSET * nbody
        for kk in range(_NSET):
            gdrain(sets[kk])
            wbi(jlast + kk, sets[kk])
        wdrain(sets[0])
        gi(NCHUNK - 1, sets[0])
        for kk in range(1, _NSET):
            wdrain(sets[kk])
        gdrain(sets[0])
        wbi(NCHUNK - 1, sets[0])
        wdrain(sets[0])

    return k(atom, src3, dst3)


_MSG_BLK = 2560


def _tc_message(xi, xj, ef, wfi, wfj, wfe, bfv, wsi, wsj, wse, bsv):
    def body(xi_ref, xj_ref, ef_ref, wfi_ref, wfj_ref, wfe_ref, bf_ref,
             wsi_ref, wsj_ref, wse_ref, bs_ref, out_ref):
        xi_b = xi_ref[...]
        xj_b = xj_ref[...]
        ef_b = ef_ref[...]
        dot = functools.partial(jnp.dot, preferred_element_type=jnp.float32)
        pf = (dot(xi_b, wfi_ref[...]) + dot(xj_b, wfj_ref[...])
              + dot(ef_b, wfe_ref[...]) + bf_ref[...])
        ps = (dot(xi_b, wsi_ref[...]) + dot(xj_b, wsj_ref[...])
              + dot(ef_b, wse_ref[...]) + bs_ref[...])
        sp = jnp.maximum(ps, 0.0) + jnp.log1p(jnp.exp(-jnp.abs(ps)))
        out_ref[...] = jax.nn.sigmoid(pf) * sp

    full = lambda shape: pl.BlockSpec(shape, lambda i: (0, 0))
    return pl.pallas_call(
        body,
        grid=(N_EDGES // _MSG_BLK,),
        in_specs=[
            pl.BlockSpec((_MSG_BLK, D), lambda i: (i, 0)),
            pl.BlockSpec((_MSG_BLK, D), lambda i: (i, 0)),
            pl.BlockSpec((_MSG_BLK, D_EDGE), lambda i: (i, 0)),
            full((D, D)), full((D, D)), full((D_EDGE, D)), full((1, D)),
            full((D, D)), full((D, D)), full((D_EDGE, D)), full((1, D)),
        ],
        out_specs=pl.BlockSpec((_MSG_BLK, D), lambda i: (i, 0)),
        out_shape=jax.ShapeDtypeStruct((N_EDGES, D), jnp.float32),
        compiler_params=pltpu.CompilerParams(
            dimension_semantics=("arbitrary",)),
    )(xi, xj, ef, wfi, wfj, wfe, bfv, wsi, wsj, wse, bsv)


def _sc_scatter_add(msg, dst3):
    @functools.partial(
        pl.kernel,
        out_type=jax.ShapeDtypeStruct((NC, N_NODES, D), jnp.float32),
        mesh=plsc.VectorSubcoreMesh(**_SC_MESH),
        scratch_types=[
            pltpu.VMEM((CHUNK, D), jnp.float32),
            pltpu.VMEM((CHUNK, D), jnp.float32),
            pltpu.VMEM((NCHUNK, CHUNK), jnp.int32),
            pltpu.VMEM((Z_ROWS, D), jnp.float32),
            pltpu.VMEM_SHARED((N_NODES, D), jnp.float32),
            pltpu.SemaphoreType.DMA,
            pltpu.SemaphoreType.DMA,
        ],
    )
    def k(msg_hbm, dst_hbm, out_hbm, m_a, m_b, idxbuf, zbuf, agg,
          sem_a, sem_b):
        c = lax.axis_index("c")
        s = lax.axis_index("s")
        wid = s * NC + c

        zero = jnp.zeros((16,), jnp.float32)

        def zrow(r, _):
            for cc in range(D // 16):
                zbuf[r, pl.ds(cc * 16, 16)] = zero
            return 0

        lax.fori_loop(0, Z_ROWS, zrow, 0)
        for t in range(STRIPE // Z_ROWS):
            pltpu.sync_copy(zbuf, agg.at[pl.ds(s * STRIPE + t * Z_ROWS, Z_ROWS)])

        @pl.when(s == NS - 1)
        def _zero_tail():
            pltpu.sync_copy(zbuf.at[pl.ds(0, STRIPE_REM)],
                            agg.at[pl.ds(NS * STRIPE, STRIPE_REM)])

        plsc.subcore_barrier()

        pltpu.sync_copy(dst_hbm.at[wid], idxbuf)
        base = wid * EPW

        def issue(j, buf, sem):
            pltpu.async_copy(msg_hbm.at[pl.ds(base + j * CHUNK, CHUNK)],
                             buf, sem)

        def drain(buf, sem):
            pltpu.make_async_copy(msg_hbm.at[pl.ds(0, CHUNK)], buf, sem).wait()

        def scat(j, buf):
            pltpu.sync_copy(buf, agg.at[idxbuf.at[j]], add=True)

        issue(0, m_a, sem_a)

        def body(t, _):
            j = 2 * t
            issue(j + 1, m_b, sem_b)
            drain(m_a, sem_a)
            scat(j, m_a)

            @pl.when(j + 2 < NCHUNK)
            def _():
                issue(j + 2, m_a, sem_a)

            drain(m_b, sem_b)
            scat(j + 1, m_b)
            return 0

        lax.fori_loop(0, NCHUNK // 2, body, 0)
        drain(m_a, sem_a)
        scat(NCHUNK - 1, m_a)
        plsc.subcore_barrier()

        pltpu.sync_copy(agg.at[pl.ds(s * STRIPE, STRIPE)],
                        out_hbm.at[c, pl.ds(s * STRIPE, STRIPE)])

        @pl.when(s == NS - 1)
        def _flush_tail():
            pltpu.sync_copy(agg.at[pl.ds(NS * STRIPE, STRIPE_REM)],
                            out_hbm.at[c, pl.ds(NS * STRIPE, STRIPE_REM)])

    return k(msg, dst3)


def _tc_node_update(partials, atom, w1ab):
    def body(p_ref, atom_ref, w1ab_ref, out_ref, q_ref):
        p = p_ref[...]
        ao = p[0] + p[1] + atom_ref[...]
        out_ref[...] = ao
        q_ref[...] = ao @ w1ab_ref[...]

    return pl.pallas_call(
        body,
        out_shape=[jax.ShapeDtypeStruct((N_NODES, D), jnp.float32),
                   jax.ShapeDtypeStruct((N_NODES, D), jnp.float32)],
    )(partials, atom, w1ab)


QCH = 64
NQCH = N_EDGES // QCH
QROWS = QCH * HID_PAD // D
H_ROWS = N_EDGES * HID_PAD // D


def _sc_gather_h(qtab, src, dst):
    @functools.partial(
        pl.kernel,
        out_type=jax.ShapeDtypeStruct((H_ROWS, D), jnp.float32),
        mesh=plsc.VectorSubcoreMesh(**_SC_MESH),
        scratch_types=[
            pltpu.VMEM((QCH,), jnp.int32),
            pltpu.VMEM((QCH,), jnp.int32),
            pltpu.VMEM((QCH,), jnp.int32),
            pltpu.VMEM((QCH,), jnp.int32),
            pltpu.VMEM((QCH, D), jnp.float32),
            pltpu.VMEM((QCH, D), jnp.float32),
            pltpu.VMEM((QCH, D), jnp.float32),
            pltpu.VMEM((QCH, D), jnp.float32),
            pltpu.VMEM((QROWS, D), jnp.float32),
            pltpu.VMEM((QROWS, D), jnp.float32),
            pltpu.SemaphoreType.DMA,
            pltpu.SemaphoreType.DMA,
            pltpu.SemaphoreType.DMA,
            pltpu.SemaphoreType.DMA,
        ],
    )
    def k(q_hbm, src_hbm, dst_hbm, h_hbm,
          is_a, id_a, is_b, id_b, s_a, d_a, s_b, d_b, h_a, h_b,
          sem_a, sem_b, semw_a, semw_b):
        wid = lax.axis_index("s") * NC + lax.axis_index("c")

        def issue(g, isx, idx, bs, bd, sem):
            pltpu.sync_copy(src_hbm.at[pl.ds(g * QCH, QCH)], isx)
            pltpu.sync_copy(dst_hbm.at[pl.ds(g * QCH, QCH)], idx)
            pltpu.async_copy(q_hbm.at[isx], bs, sem)
            pltpu.async_copy(q_hbm.at[idx], bd, sem)

        def drain(bs, bd, sem):
            pltpu.make_async_copy(q_hbm.at[pl.ds(0, QCH)], bs, sem).wait()
            pltpu.make_async_copy(q_hbm.at[pl.ds(0, QCH)], bd, sem).wait()

        def addwb(g, bs, bd, hb, semw):
            for e in range(QCH):
                hb[e // 8, pl.ds((e % 8) * HID_PAD, HID_PAD)] = (
                    bs[e, pl.ds(0, HID_PAD)] + bd[e, pl.ds(HID_PAD, HID_PAD)])
            pltpu.async_copy(hb, h_hbm.at[pl.ds(g * QROWS, QROWS)], semw)

        def wdrain(hb, semw):
            pltpu.make_async_copy(hb, h_hbm.at[pl.ds(0, QROWS)], semw).wait()

        issue(wid, is_a, id_a, s_a, d_a, sem_a)

        def body(t, _):
            g0 = (2 * t) * NW + wid
            g1 = g0 + NW
            g2 = g1 + NW
            issue(g1, is_b, id_b, s_b, d_b, sem_b)
            drain(s_a, d_a, sem_a)

            @pl.when(t > 0)
            def _():
                wdrain(h_a, semw_a)

            addwb(g0, s_a, d_a, h_a, semw_a)

            @pl.when(g2 < NQCH)
            def _():
                issue(g2, is_a, id_a, s_a, d_a, sem_a)

            drain(s_b, d_b, sem_b)

            @pl.when(t > 0)
            def _():
                wdrain(h_b, semw_b)

            addwb(g1, s_b, d_b, h_b, semw_b)
            return 0

        lax.fori_loop(0, 78, body, 0)
        wdrain(h_b, semw_b)
        glast = 156 * NW + wid

        @pl.when(glast < NQCH)
        def _tail():
            drain(s_a, d_a, sem_a)
            wdrain(h_a, semw_a)
            addwb(glast, s_a, d_a, h_a, semw_a)

        wdrain(h_a, semw_a)

    return k(qtab, src, dst)


_EDGE_BLK = 2560


def _tc_edge_mlp(h, ef, w1e, b1v, w2, b2v):
    def body(h_ref, ef_ref, w1e_ref, b1_ref, w2_ref, b2_ref, out_ref):
        hp = h_ref[...] + ef_ref[...] @ w1e_ref[...] + b1_ref[...]
        hh = hp * jax.nn.sigmoid(hp)
        o = hh @ w2_ref[...] + b2_ref[...]
        out_ref[...] = o * jax.nn.sigmoid(o)

    full = lambda shape: pl.BlockSpec(shape, lambda i: (0, 0))
    return pl.pallas_call(
        body,
        grid=(N_EDGES // _EDGE_BLK,),
        in_specs=[
            pl.BlockSpec((_EDGE_BLK, HID_PAD), lambda i: (i, 0)),
            pl.BlockSpec((_EDGE_BLK, D_EDGE), lambda i: (i, 0)),
            full((D_EDGE, HID_PAD)), full((1, HID_PAD)),
            full((HID_PAD, D)), full((1, D)),
        ],
        out_specs=pl.BlockSpec((_EDGE_BLK, D), lambda i: (i, 0)),
        out_shape=jax.ShapeDtypeStruct((N_EDGES, D), jnp.float32),
        compiler_params=pltpu.CompilerParams(
            dimension_semantics=("arbitrary",)),
    )(h, ef, w1e, b1v, w2, b2v)


def kernel(atom_fea, edge_idx, edge_fea, batch, distance, edge_vec,
           Wf, bf, Ws, bs, W1, b1, W2, b2):
    src = edge_idx[0].astype(jnp.int32)
    dst = edge_idx[1].astype(jnp.int32)
    src3 = src.reshape(NW, NCHUNK, CHUNK)
    dst3 = dst.reshape(NW, NCHUNK, CHUNK)

    xj, xi = _sc_gather_xixj(atom_fea, src3, dst3)

    wfi, wfj, wfe = Wf[:, :D].T, Wf[:, D:2 * D].T, Wf[:, 2 * D:].T
    wsi, wsj, wse = Ws[:, :D].T, Ws[:, D:2 * D].T, Ws[:, 2 * D:].T
    msg = _tc_message(xi, xj, edge_fea, wfi, wfj, wfe, bf.reshape(1, D),
                      wsi, wsj, wse, bs.reshape(1, D))

    partials = _sc_scatter_add(msg, dst3)

    pad = jnp.zeros((D, HID_PAD - HID), jnp.float32)
    w1ab = jnp.concatenate(
        [W1[:, :D].T, pad, W1[:, D:2 * D].T, pad,
         jnp.zeros((D, D - 2 * HID_PAD), jnp.float32)], axis=1)
    atom_out, qtab = _tc_node_update(partials, atom_fea, w1ab)

    h = _sc_gather_h(qtab, src, dst).reshape(N_EDGES, HID_PAD)

    epad = jnp.zeros((D_EDGE, HID_PAD - HID), jnp.float32)
    w1e = jnp.concatenate([W1[:, 2 * D:].T, epad], axis=1)
    b1v = jnp.concatenate([b1, jnp.zeros((HID_PAD - HID,), jnp.float32)])
    w2 = jnp.concatenate([W2.T, jnp.zeros((HID_PAD - HID, D), jnp.float32)],
                         axis=0)
    edge_out = _tc_edge_mlp(h, edge_fea, w1e, b1v.reshape(1, HID_PAD),
                            w2, b2.reshape(1, D))
    return atom_out, edge_out


# speedup vs baseline: 1.1386x; 1.0431x over previous
import functools

import jax
import jax.numpy as jnp
import numpy as np
from jax import lax
from jax.experimental import pallas as pl
from jax.experimental.pallas import tpu as pltpu
from jax.experimental.pallas import tpu_sc as plsc

N_NODES = 10000
N_EDGES = 320000
D = 128
D_EDGE = 16
HID = 14
HID_PAD = 16

NC = 2
NS = 16
NW = NC * NS
EPW = N_EDGES // NW
CHUNK = 80
NCHUNK = EPW // CHUNK
STRIPE = 624
STRIPE_REM = N_NODES - NS * STRIPE
Z_ROWS = 16

_SC_MESH = dict(core_axis_name="c", subcore_axis_name="s")


_NSET = 4


def _sc_gather_xixj(atom, src3, dst3):
    @functools.partial(
        pl.kernel,
        out_type=[jax.ShapeDtypeStruct((N_EDGES, D), jnp.float32),
                  jax.ShapeDtypeStruct((N_EDGES, D), jnp.float32)],
        mesh=plsc.VectorSubcoreMesh(**_SC_MESH),
        scratch_types=(
            [pltpu.VMEM((NCHUNK, CHUNK), jnp.int32)] * 2
            + [pltpu.VMEM((CHUNK, D), jnp.float32)] * (2 * _NSET)
            + [pltpu.SemaphoreType.DMA] * (2 * _NSET)
        ),
    )
    def k(atom_hbm, src_hbm, dst_hbm, xj_hbm, xi_hbm, idx_s, idx_d, *rest):
        bufs = rest[:2 * _NSET]
        gsems = rest[2 * _NSET:3 * _NSET]
        wsems = rest[3 * _NSET:]
        sets = [(bufs[2 * i], bufs[2 * i + 1], gsems[i], wsems[i])
                for i in range(_NSET)]
        wid = lax.axis_index("s") * NC + lax.axis_index("c")
        base = wid * EPW
        pltpu.sync_copy(src_hbm.at[wid], idx_s)
        pltpu.sync_copy(dst_hbm.at[wid], idx_d)

        def gi(j, S):
            bs, bd, gs, _ = S
            pltpu.async_copy(atom_hbm.at[idx_s.at[j]], bs, gs)
            pltpu.async_copy(atom_hbm.at[idx_d.at[j]], bd, gs)

        def gdrain(S):
            bs, bd, gs, _ = S
            pltpu.make_async_copy(atom_hbm.at[pl.ds(0, CHUNK)], bs, gs).wait()
            pltpu.make_async_copy(atom_hbm.at[pl.ds(0, CHUNK)], bd, gs).wait()

        def wbi(j, S):
            bs, bd, _, ws = S
            off = base + j * CHUNK
            pltpu.async_copy(bs, xj_hbm.at[pl.ds(off, CHUNK)], ws)
            pltpu.async_copy(bd, xi_hbm.at[pl.ds(off, CHUNK)], ws)

        def wdrain(S):
            bs, bd, _, ws = S
            pltpu.make_async_copy(bs, xj_hbm.at[pl.ds(0, CHUNK)], ws).wait()
            pltpu.make_async_copy(bd, xi_hbm.at[pl.ds(0, CHUNK)], ws).wait()

        for kk in range(_NSET):
            gi(kk, sets[kk])

        nbody = (NCHUNK - 1) // _NSET - 1

        def body(t, _):
            j = _NSET * t
            for kk in range(_NSET):
                gdrain(sets[kk])
                wbi(j + kk, sets[kk])
            for kk in range(_NSET):
                wdrain(sets[kk])
                gi(j + _NSET + kk, sets[kk])
            return 0

        lax.fori_loop(0, nbody, body, 0)
        jlast = _NSET * nbody
        for kk in range(_NSET):
            gdrain(sets[kk])
            wbi(jlast + kk, sets[kk])
        wdrain(sets[0])
        gi(NCHUNK - 1, sets[0])
        for kk in range(1, _NSET):
            wdrain(sets[kk])
        gdrain(sets[0])
        wbi(NCHUNK - 1, sets[0])
        wdrain(sets[0])

    return k(atom, src3, dst3)


_MSG_BLK = 4000


def _tc_message(xi, xj, ef, wfi, wfj, wfe, bfv, wsi, wsj, wse, bsv):
    def body(xi_ref, xj_ref, ef_ref, wfi_ref, wfj_ref, wfe_ref, bf_ref,
             wsi_ref, wsj_ref, wse_ref, bs_ref, out_ref):
        bft = jnp.bfloat16
        xi_b = xi_ref[...].astype(bft)
        xj_b = xj_ref[...].astype(bft)
        ef_b = ef_ref[...].astype(bft)
        dot = functools.partial(jnp.dot, preferred_element_type=jnp.float32)
        pf = (dot(xi_b, wfi_ref[...].astype(bft))
              + dot(xj_b, wfj_ref[...].astype(bft))
              + dot(ef_b, wfe_ref[...].astype(bft)) + bf_ref[...])
        ps = (dot(xi_b, wsi_ref[...].astype(bft))
              + dot(xj_b, wsj_ref[...].astype(bft))
              + dot(ef_b, wse_ref[...].astype(bft)) + bs_ref[...])
        sp = jnp.maximum(ps, 0.0) + jnp.log1p(jnp.exp(-jnp.abs(ps)))
        out_ref[...] = jax.nn.sigmoid(pf) * sp

    full = lambda shape: pl.BlockSpec(shape, lambda i: (0, 0))
    return pl.pallas_call(
        body,
        grid=(N_EDGES // _MSG_BLK,),
        in_specs=[
            pl.BlockSpec((_MSG_BLK, D), lambda i: (i, 0)),
            pl.BlockSpec((_MSG_BLK, D), lambda i: (i, 0)),
            pl.BlockSpec((_MSG_BLK, D_EDGE), lambda i: (i, 0)),
            full((D, D)), full((D, D)), full((D_EDGE, D)), full((1, D)),
            full((D, D)), full((D, D)), full((D_EDGE, D)), full((1, D)),
        ],
        out_specs=pl.BlockSpec((_MSG_BLK, D), lambda i: (i, 0)),
        out_shape=jax.ShapeDtypeStruct((N_EDGES, D), jnp.float32),
        compiler_params=pltpu.CompilerParams(
            dimension_semantics=("arbitrary",)),
    )(xi, xj, ef, wfi, wfj, wfe, bfv, wsi, wsj, wse, bsv)


def _sc_scatter_add(msg, dst3):
    @functools.partial(
        pl.kernel,
        out_type=jax.ShapeDtypeStruct((NC, N_NODES, D), jnp.float32),
        mesh=plsc.VectorSubcoreMesh(**_SC_MESH),
        scratch_types=[
            pltpu.VMEM((CHUNK, D), jnp.float32),
            pltpu.VMEM((CHUNK, D), jnp.float32),
            pltpu.VMEM((NCHUNK, CHUNK), jnp.int32),
            pltpu.VMEM((Z_ROWS, D), jnp.float32),
            pltpu.VMEM_SHARED((N_NODES, D), jnp.float32),
            pltpu.SemaphoreType.DMA,
            pltpu.SemaphoreType.DMA,
        ],
    )
    def k(msg_hbm, dst_hbm, out_hbm, m_a, m_b, idxbuf, zbuf, agg,
          sem_a, sem_b):
        c = lax.axis_index("c")
        s = lax.axis_index("s")
        wid = s * NC + c

        zero = jnp.zeros((16,), jnp.float32)

        def zrow(r, _):
            for cc in range(D // 16):
                zbuf[r, pl.ds(cc * 16, 16)] = zero
            return 0

        lax.fori_loop(0, Z_ROWS, zrow, 0)
        for t in range(STRIPE // Z_ROWS):
            pltpu.sync_copy(zbuf, agg.at[pl.ds(s * STRIPE + t * Z_ROWS, Z_ROWS)])

        @pl.when(s == NS - 1)
        def _zero_tail():
            pltpu.sync_copy(zbuf.at[pl.ds(0, STRIPE_REM)],
                            agg.at[pl.ds(NS * STRIPE, STRIPE_REM)])

        plsc.subcore_barrier()

        pltpu.sync_copy(dst_hbm.at[wid], idxbuf)
        base = wid * EPW

        def issue(j, buf, sem):
            pltpu.async_copy(msg_hbm.at[pl.ds(base + j * CHUNK, CHUNK)],
                             buf, sem)

        def drain(buf, sem):
            pltpu.make_async_copy(msg_hbm.at[pl.ds(0, CHUNK)], buf, sem).wait()

        def scat(j, buf):
            pltpu.sync_copy(buf, agg.at[idxbuf.at[j]], add=True)

        issue(0, m_a, sem_a)

        def body(t, _):
            j = 2 * t
            issue(j + 1, m_b, sem_b)
            drain(m_a, sem_a)
            scat(j, m_a)

            @pl.when(j + 2 < NCHUNK)
            def _():
                issue(j + 2, m_a, sem_a)

            drain(m_b, sem_b)
            scat(j + 1, m_b)
            return 0

        lax.fori_loop(0, NCHUNK // 2, body, 0)
        drain(m_a, sem_a)
        scat(NCHUNK - 1, m_a)
        plsc.subcore_barrier()

        pltpu.sync_copy(agg.at[pl.ds(s * STRIPE, STRIPE)],
                        out_hbm.at[c, pl.ds(s * STRIPE, STRIPE)])

        @pl.when(s == NS - 1)
        def _flush_tail():
            pltpu.sync_copy(agg.at[pl.ds(NS * STRIPE, STRIPE_REM)],
                            out_hbm.at[c, pl.ds(NS * STRIPE, STRIPE_REM)])

    return k(msg, dst3)


def _tc_node_update(partials, atom, w1ab):
    def body(p_ref, atom_ref, w1ab_ref, out_ref, q_ref):
        p = p_ref[...]
        ao = p[0] + p[1] + atom_ref[...]
        out_ref[...] = ao
        q_ref[...] = ao @ w1ab_ref[...]

    return pl.pallas_call(
        body,
        out_shape=[jax.ShapeDtypeStruct((N_NODES, D), jnp.float32),
                   jax.ShapeDtypeStruct((N_NODES, D), jnp.float32)],
    )(partials, atom, w1ab)


QCH = 64
NQCH = N_EDGES // QCH
QROWS = QCH * HID_PAD // D
H_ROWS = N_EDGES * HID_PAD // D


def _sc_gather_h(qtab, src, dst):
    @functools.partial(
        pl.kernel,
        out_type=jax.ShapeDtypeStruct((H_ROWS, D), jnp.float32),
        mesh=plsc.VectorSubcoreMesh(**_SC_MESH),
        scratch_types=[
            pltpu.VMEM((QCH,), jnp.int32),
            pltpu.VMEM((QCH,), jnp.int32),
            pltpu.VMEM((QCH,), jnp.int32),
            pltpu.VMEM((QCH,), jnp.int32),
            pltpu.VMEM((QCH, D), jnp.float32),
            pltpu.VMEM((QCH, D), jnp.float32),
            pltpu.VMEM((QCH, D), jnp.float32),
            pltpu.VMEM((QCH, D), jnp.float32),
            pltpu.VMEM((QROWS, D), jnp.float32),
            pltpu.VMEM((QROWS, D), jnp.float32),
            pltpu.SemaphoreType.DMA,
            pltpu.SemaphoreType.DMA,
            pltpu.SemaphoreType.DMA,
            pltpu.SemaphoreType.DMA,
        ],
    )
    def k(q_hbm, src_hbm, dst_hbm, h_hbm,
          is_a, id_a, is_b, id_b, s_a, d_a, s_b, d_b, h_a, h_b,
          sem_a, sem_b, semw_a, semw_b):
        wid = lax.axis_index("s") * NC + lax.axis_index("c")

        def issue(g, isx, idx, bs, bd, sem):
            pltpu.sync_copy(src_hbm.at[pl.ds(g * QCH, QCH)], isx)
            pltpu.sync_copy(dst_hbm.at[pl.ds(g * QCH, QCH)], idx)
            pltpu.async_copy(q_hbm.at[isx], bs, sem)
            pltpu.async_copy(q_hbm.at[idx], bd, sem)

        def drain(bs, bd, sem):
            pltpu.make_async_copy(q_hbm.at[pl.ds(0, QCH)], bs, sem).wait()
            pltpu.make_async_copy(q_hbm.at[pl.ds(0, QCH)], bd, sem).wait()

        def addwb(g, bs, bd, hb, semw):
            for e in range(QCH):
                hb[e // 8, pl.ds((e % 8) * HID_PAD, HID_PAD)] = (
                    bs[e, pl.ds(0, HID_PAD)] + bd[e, pl.ds(HID_PAD, HID_PAD)])
            pltpu.async_copy(hb, h_hbm.at[pl.ds(g * QROWS, QROWS)], semw)

        def wdrain(hb, semw):
            pltpu.make_async_copy(hb, h_hbm.at[pl.ds(0, QROWS)], semw).wait()

        issue(wid, is_a, id_a, s_a, d_a, sem_a)

        def body(t, _):
            g0 = (2 * t) * NW + wid
            g1 = g0 + NW
            g2 = g1 + NW
            issue(g1, is_b, id_b, s_b, d_b, sem_b)
            drain(s_a, d_a, sem_a)

            @pl.when(t > 0)
            def _():
                wdrain(h_a, semw_a)

            addwb(g0, s_a, d_a, h_a, semw_a)

            @pl.when(g2 < NQCH)
            def _():
                issue(g2, is_a, id_a, s_a, d_a, sem_a)

            drain(s_b, d_b, sem_b)

            @pl.when(t > 0)
            def _():
                wdrain(h_b, semw_b)

            addwb(g1, s_b, d_b, h_b, semw_b)
            return 0

        lax.fori_loop(0, 78, body, 0)
        wdrain(h_b, semw_b)
        glast = 156 * NW + wid

        @pl.when(glast < NQCH)
        def _tail():
            drain(s_a, d_a, sem_a)
            wdrain(h_a, semw_a)
            addwb(glast, s_a, d_a, h_a, semw_a)

        wdrain(h_a, semw_a)

    return k(qtab, src, dst)


_EDGE_BLK = 4000


def _tc_edge_mlp(h, ef, w1e, b1v, w2, b2v):
    def body(h_ref, ef_ref, w1e_ref, b1_ref, w2_ref, b2_ref, out_ref):
        hp = h_ref[...] + ef_ref[...] @ w1e_ref[...] + b1_ref[...]
        hh = hp * jax.nn.sigmoid(hp)
        o = hh @ w2_ref[...] + b2_ref[...]
        out_ref[...] = o * jax.nn.sigmoid(o)

    full = lambda shape: pl.BlockSpec(shape, lambda i: (0, 0))
    return pl.pallas_call(
        body,
        grid=(N_EDGES // _EDGE_BLK,),
        in_specs=[
            pl.BlockSpec((_EDGE_BLK, HID_PAD), lambda i: (i, 0)),
            pl.BlockSpec((_EDGE_BLK, D_EDGE), lambda i: (i, 0)),
            full((D_EDGE, HID_PAD)), full((1, HID_PAD)),
            full((HID_PAD, D)), full((1, D)),
        ],
        out_specs=pl.BlockSpec((_EDGE_BLK, D), lambda i: (i, 0)),
        out_shape=jax.ShapeDtypeStruct((N_EDGES, D), jnp.float32),
        compiler_params=pltpu.CompilerParams(
            dimension_semantics=("arbitrary",)),
    )(h, ef, w1e, b1v, w2, b2v)


def kernel(atom_fea, edge_idx, edge_fea, batch, distance, edge_vec,
           Wf, bf, Ws, bs, W1, b1, W2, b2):
    src = edge_idx[0].astype(jnp.int32)
    dst = edge_idx[1].astype(jnp.int32)
    src3 = src.reshape(NW, NCHUNK, CHUNK)
    dst3 = dst.reshape(NW, NCHUNK, CHUNK)

    xj, xi = _sc_gather_xixj(atom_fea, src3, dst3)

    wfi, wfj, wfe = Wf[:, :D].T, Wf[:, D:2 * D].T, Wf[:, 2 * D:].T
    wsi, wsj, wse = Ws[:, :D].T, Ws[:, D:2 * D].T, Ws[:, 2 * D:].T
    msg = _tc_message(xi, xj, edge_fea, wfi, wfj, wfe, bf.reshape(1, D),
                      wsi, wsj, wse, bs.reshape(1, D))

    partials = _sc_scatter_add(msg, dst3)

    pad = jnp.zeros((D, HID_PAD - HID), jnp.float32)
    w1ab = jnp.concatenate(
        [W1[:, :D].T, pad, W1[:, D:2 * D].T, pad,
         jnp.zeros((D, D - 2 * HID_PAD), jnp.float32)], axis=1)
    atom_out, qtab = _tc_node_update(partials, atom_fea, w1ab)

    h = _sc_gather_h(qtab, src, dst).reshape(N_EDGES, HID_PAD)

    epad = jnp.zeros((D_EDGE, HID_PAD - HID), jnp.float32)
    w1e = jnp.concatenate([W1[:, 2 * D:].T, epad], axis=1)
    b1v = jnp.concatenate([b1, jnp.zeros((HID_PAD - HID,), jnp.float32)])
    w2 = jnp.concatenate([W2.T, jnp.zeros((HID_PAD - HID, D), jnp.float32)],
                         axis=0)
    edge_out = _tc_edge_mlp(h, edge_fea, w1e, b1v.reshape(1, HID_PAD),
                            w2, b2.reshape(1, D))
    return atom_out, edge_out


# speedup vs baseline: 1.1389x; 1.0003x over previous
import functools

import jax
import jax.numpy as jnp
from jax import lax
from jax.experimental import pallas as pl
from jax.experimental.pallas import tpu as pltpu
from jax.experimental.pallas import tpu_sc as plsc

N_NODES = 10000
N_EDGES = 320000
D = 128
D_EDGE = 16
HID = 14
HID_PAD = 16

NC = 2
NS = 16
NW = NC * NS
EPW = N_EDGES // NW
CHUNK = 80
NCHUNK = EPW // CHUNK
STRIPE = 624
STRIPE_REM = N_NODES - NS * STRIPE
Z_ROWS = 16

_SC_MESH = dict(core_axis_name="c", subcore_axis_name="s")


_NSET = 4


def _sc_gather_xixj(atom, src3, dst3):
    @functools.partial(
        pl.kernel,
        out_type=[jax.ShapeDtypeStruct((N_EDGES, D), jnp.float32),
                  jax.ShapeDtypeStruct((N_EDGES, D), jnp.float32)],
        mesh=plsc.VectorSubcoreMesh(**_SC_MESH),
        scratch_types=(
            [pltpu.VMEM((NCHUNK, CHUNK), jnp.int32)] * 2
            + [pltpu.VMEM((CHUNK, D), jnp.float32)] * (2 * _NSET)
            + [pltpu.SemaphoreType.DMA] * (2 * _NSET)
        ),
    )
    def k(atom_hbm, src_hbm, dst_hbm, xj_hbm, xi_hbm, idx_s, idx_d, *rest):
        bufs = rest[:2 * _NSET]
        gsems = rest[2 * _NSET:3 * _NSET]
        wsems = rest[3 * _NSET:]
        sets = [(bufs[2 * i], bufs[2 * i + 1], gsems[i], wsems[i])
                for i in range(_NSET)]
        wid = lax.axis_index("s") * NC + lax.axis_index("c")
        base = wid * EPW
        pltpu.sync_copy(src_hbm.at[wid], idx_s)
        pltpu.sync_copy(dst_hbm.at[wid], idx_d)

        def gi(j, S):
            bs, bd, gs, _ = S
            pltpu.async_copy(atom_hbm.at[idx_s.at[j]], bs, gs)
            pltpu.async_copy(atom_hbm.at[idx_d.at[j]], bd, gs)

        def gdrain(S):
            bs, bd, gs, _ = S
            pltpu.make_async_copy(atom_hbm.at[pl.ds(0, CHUNK)], bs, gs).wait()
            pltpu.make_async_copy(atom_hbm.at[pl.ds(0, CHUNK)], bd, gs).wait()

        def wbi(j, S):
            bs, bd, _, ws = S
            off = base + j * CHUNK
            pltpu.async_copy(bs, xj_hbm.at[pl.ds(off, CHUNK)], ws)
            pltpu.async_copy(bd, xi_hbm.at[pl.ds(off, CHUNK)], ws)

        def wdrain(S):
            bs, bd, _, ws = S
            pltpu.make_async_copy(bs, xj_hbm.at[pl.ds(0, CHUNK)], ws).wait()
            pltpu.make_async_copy(bd, xi_hbm.at[pl.ds(0, CHUNK)], ws).wait()

        for kk in range(_NSET):
            gi(kk, sets[kk])

        nbody = (NCHUNK - 1) // _NSET - 1

        def body(t, _):
            j = _NSET * t
            for kk in range(_NSET):
                gdrain(sets[kk])
                wbi(j + kk, sets[kk])
            for kk in range(_NSET):
                wdrain(sets[kk])
                gi(j + _NSET + kk, sets[kk])
            return 0

        lax.fori_loop(0, nbody, body, 0)
        jlast = _NSET * nbody
        for kk in range(_NSET):
            gdrain(sets[kk])
            wbi(jlast + kk, sets[kk])
        wdrain(sets[0])
        gi(NCHUNK - 1, sets[0])
        for kk in range(1, _NSET):
            wdrain(sets[kk])
        gdrain(sets[0])
        wbi(NCHUNK - 1, sets[0])
        wdrain(sets[0])

    return k(atom, src3, dst3)


_MSG_BLK = 4000


def _tc_message(xi, xj, ef, wfi, wfj, wfe, bfv, wsi, wsj, wse, bsv):
    def body(xi_ref, xj_ref, ef_ref, wfi_ref, wfj_ref, wfe_ref, bf_ref,
             wsi_ref, wsj_ref, wse_ref, bs_ref, out_ref):
        bft = jnp.bfloat16
        xi_b = xi_ref[...].astype(bft)
        xj_b = xj_ref[...].astype(bft)
        ef_b = ef_ref[...].astype(bft)
        dot = functools.partial(jnp.dot, preferred_element_type=jnp.float32)
        pf = (dot(xi_b, wfi_ref[...].astype(bft))
              + dot(xj_b, wfj_ref[...].astype(bft))
              + dot(ef_b, wfe_ref[...].astype(bft)) + bf_ref[...])
        ps = (dot(xi_b, wsi_ref[...].astype(bft))
              + dot(xj_b, wsj_ref[...].astype(bft))
              + dot(ef_b, wse_ref[...].astype(bft)) + bs_ref[...])
        sp = jnp.maximum(ps, 0.0) + jnp.log1p(jnp.exp(-jnp.abs(ps)))
        out_ref[...] = jax.nn.sigmoid(pf) * sp

    full = lambda shape: pl.BlockSpec(shape, lambda i: (0, 0))
    return pl.pallas_call(
        body,
        grid=(N_EDGES // _MSG_BLK,),
        in_specs=[
            pl.BlockSpec((_MSG_BLK, D), lambda i: (i, 0)),
            pl.BlockSpec((_MSG_BLK, D), lambda i: (i, 0)),
            pl.BlockSpec((_MSG_BLK, D_EDGE), lambda i: (i, 0)),
            full((D, D)), full((D, D)), full((D_EDGE, D)), full((1, D)),
            full((D, D)), full((D, D)), full((D_EDGE, D)), full((1, D)),
        ],
        out_specs=pl.BlockSpec((_MSG_BLK, D), lambda i: (i, 0)),
        out_shape=jax.ShapeDtypeStruct((N_EDGES, D), jnp.float32),
        compiler_params=pltpu.CompilerParams(
            dimension_semantics=("arbitrary",)),
    )(xi, xj, ef, wfi, wfj, wfe, bfv, wsi, wsj, wse, bsv)


def _sc_scatter_add(msg, dst3):
    @functools.partial(
        pl.kernel,
        out_type=jax.ShapeDtypeStruct((NC, N_NODES, D), jnp.float32),
        mesh=plsc.VectorSubcoreMesh(**_SC_MESH),
        scratch_types=[
            pltpu.VMEM((CHUNK, D), jnp.float32),
            pltpu.VMEM((CHUNK, D), jnp.float32),
            pltpu.VMEM((NCHUNK, CHUNK), jnp.int32),
            pltpu.VMEM((Z_ROWS, D), jnp.float32),
            pltpu.VMEM_SHARED((N_NODES, D), jnp.float32),
            pltpu.SemaphoreType.DMA,
            pltpu.SemaphoreType.DMA,
        ],
    )
    def k(msg_hbm, dst_hbm, out_hbm, m_a, m_b, idxbuf, zbuf, agg,
          sem_a, sem_b):
        c = lax.axis_index("c")
        s = lax.axis_index("s")
        wid = s * NC + c

        zero = jnp.zeros((16,), jnp.float32)

        def zrow(r, _):
            for cc in range(D // 16):
                zbuf[r, pl.ds(cc * 16, 16)] = zero
            return 0

        lax.fori_loop(0, Z_ROWS, zrow, 0)
        for t in range(STRIPE // Z_ROWS):
            pltpu.sync_copy(zbuf, agg.at[pl.ds(s * STRIPE + t * Z_ROWS, Z_ROWS)])

        @pl.when(s == NS - 1)
        def _zero_tail():
            pltpu.sync_copy(zbuf.at[pl.ds(0, STRIPE_REM)],
                            agg.at[pl.ds(NS * STRIPE, STRIPE_REM)])

        plsc.subcore_barrier()

        pltpu.sync_copy(dst_hbm.at[wid], idxbuf)
        base = wid * EPW

        def issue(j, buf, sem):
            pltpu.async_copy(msg_hbm.at[pl.ds(base + j * CHUNK, CHUNK)],
                             buf, sem)

        def drain(buf, sem):
            pltpu.make_async_copy(msg_hbm.at[pl.ds(0, CHUNK)], buf, sem).wait()

        def scat(j, buf):
            pltpu.sync_copy(buf, agg.at[idxbuf.at[j]], add=True)

        issue(0, m_a, sem_a)

        def body(t, _):
            j = 2 * t
            issue(j + 1, m_b, sem_b)
            drain(m_a, sem_a)
            scat(j, m_a)

            @pl.when(j + 2 < NCHUNK)
            def _():
                issue(j + 2, m_a, sem_a)

            drain(m_b, sem_b)
            scat(j + 1, m_b)
            return 0

        lax.fori_loop(0, NCHUNK // 2, body, 0)
        drain(m_a, sem_a)
        scat(NCHUNK - 1, m_a)
        plsc.subcore_barrier()

        pltpu.sync_copy(agg.at[pl.ds(s * STRIPE, STRIPE)],
                        out_hbm.at[c, pl.ds(s * STRIPE, STRIPE)])

        @pl.when(s == NS - 1)
        def _flush_tail():
            pltpu.sync_copy(agg.at[pl.ds(NS * STRIPE, STRIPE_REM)],
                            out_hbm.at[c, pl.ds(NS * STRIPE, STRIPE_REM)])

    return k(msg, dst3)


def _tc_node_update(partials, atom, w1ab):
    def body(p_ref, atom_ref, w1ab_ref, out_ref, q_ref):
        p = p_ref[...]
        ao = p[0] + p[1] + atom_ref[...]
        out_ref[...] = ao
        q_ref[...] = ao @ w1ab_ref[...]

    return pl.pallas_call(
        body,
        out_shape=[jax.ShapeDtypeStruct((N_NODES, D), jnp.float32),
                   jax.ShapeDtypeStruct((N_NODES, D), jnp.float32)],
    )(partials, atom, w1ab)


QCH = 64
NQCH = N_EDGES // QCH
QROWS = QCH * HID_PAD // D
H_ROWS = N_EDGES * HID_PAD // D


def _sc_gather_h(qtab, src, dst):
    @functools.partial(
        pl.kernel,
        out_type=jax.ShapeDtypeStruct((H_ROWS, D), jnp.float32),
        mesh=plsc.VectorSubcoreMesh(**_SC_MESH),
        scratch_types=[
            pltpu.VMEM((QCH,), jnp.int32),
            pltpu.VMEM((QCH,), jnp.int32),
            pltpu.VMEM((QCH,), jnp.int32),
            pltpu.VMEM((QCH,), jnp.int32),
            pltpu.VMEM((QCH, D), jnp.float32),
            pltpu.VMEM((QCH, D), jnp.float32),
            pltpu.VMEM((QCH, D), jnp.float32),
            pltpu.VMEM((QCH, D), jnp.float32),
            pltpu.VMEM((QROWS, D), jnp.float32),
            pltpu.VMEM((QROWS, D), jnp.float32),
            pltpu.SemaphoreType.DMA,
            pltpu.SemaphoreType.DMA,
            pltpu.SemaphoreType.DMA,
            pltpu.SemaphoreType.DMA,
        ],
    )
    def k(q_hbm, src_hbm, dst_hbm, h_hbm,
          is_a, id_a, is_b, id_b, s_a, d_a, s_b, d_b, h_a, h_b,
          sem_a, sem_b, semw_a, semw_b):
        wid = lax.axis_index("s") * NC + lax.axis_index("c")

        def issue(g, isx, idx, bs, bd, sem):
            pltpu.sync_copy(src_hbm.at[pl.ds(g * QCH, QCH)], isx)
            pltpu.sync_copy(dst_hbm.at[pl.ds(g * QCH, QCH)], idx)
            pltpu.async_copy(q_hbm.at[isx], bs, sem)
            pltpu.async_copy(q_hbm.at[idx], bd, sem)

        def drain(bs, bd, sem):
            pltpu.make_async_copy(q_hbm.at[pl.ds(0, QCH)], bs, sem).wait()
            pltpu.make_async_copy(q_hbm.at[pl.ds(0, QCH)], bd, sem).wait()

        def addwb(g, bs, bd, hb, semw):
            for e in range(QCH):
                hb[e // 8, pl.ds((e % 8) * HID_PAD, HID_PAD)] = (
                    bs[e, pl.ds(0, HID_PAD)] + bd[e, pl.ds(HID_PAD, HID_PAD)])
            pltpu.async_copy(hb, h_hbm.at[pl.ds(g * QROWS, QROWS)], semw)

        def wdrain(hb, semw):
            pltpu.make_async_copy(hb, h_hbm.at[pl.ds(0, QROWS)], semw).wait()

        issue(wid, is_a, id_a, s_a, d_a, sem_a)

        def body(t, _):
            g0 = (2 * t) * NW + wid
            g1 = g0 + NW
            g2 = g1 + NW
            issue(g1, is_b, id_b, s_b, d_b, sem_b)
            drain(s_a, d_a, sem_a)

            @pl.when(t > 0)
            def _():
                wdrain(h_a, semw_a)

            addwb(g0, s_a, d_a, h_a, semw_a)

            @pl.when(g2 < NQCH)
            def _():
                issue(g2, is_a, id_a, s_a, d_a, sem_a)

            drain(s_b, d_b, sem_b)

            @pl.when(t > 0)
            def _():
                wdrain(h_b, semw_b)

            addwb(g1, s_b, d_b, h_b, semw_b)
            return 0

        lax.fori_loop(0, 78, body, 0)
        wdrain(h_b, semw_b)
        glast = 156 * NW + wid

        @pl.when(glast < NQCH)
        def _tail():
            drain(s_a, d_a, sem_a)
            wdrain(h_a, semw_a)
            addwb(glast, s_a, d_a, h_a, semw_a)

        wdrain(h_a, semw_a)

    return k(qtab, src, dst)


_EDGE_BLK = 4000


def _tc_edge_mlp(h, ef, w1e, b1v, w2, b2v):
    def body(h_ref, ef_ref, w1e_ref, b1_ref, w2_ref, b2_ref, out_ref):
        hp = h_ref[...] + ef_ref[...] @ w1e_ref[...] + b1_ref[...]
        hh = hp * jax.nn.sigmoid(hp)
        o = hh @ w2_ref[...] + b2_ref[...]
        out_ref[...] = o * jax.nn.sigmoid(o)

    full = lambda shape: pl.BlockSpec(shape, lambda i: (0, 0))
    return pl.pallas_call(
        body,
        grid=(N_EDGES // _EDGE_BLK,),
        in_specs=[
            pl.BlockSpec((_EDGE_BLK, HID_PAD), lambda i: (i, 0)),
            pl.BlockSpec((_EDGE_BLK, D_EDGE), lambda i: (i, 0)),
            full((D_EDGE, HID_PAD)), full((1, HID_PAD)),
            full((HID_PAD, D)), full((1, D)),
        ],
        out_specs=pl.BlockSpec((_EDGE_BLK, D), lambda i: (i, 0)),
        out_shape=jax.ShapeDtypeStruct((N_EDGES, D), jnp.float32),
        compiler_params=pltpu.CompilerParams(
            dimension_semantics=("arbitrary",)),
    )(h, ef, w1e, b1v, w2, b2v)


def kernel(atom_fea, edge_idx, edge_fea, batch, distance, edge_vec,
           Wf, bf, Ws, bs, W1, b1, W2, b2):
    src = edge_idx[0].astype(jnp.int32)
    dst = edge_idx[1].astype(jnp.int32)
    src3 = src.reshape(NW, NCHUNK, CHUNK)
    dst3 = dst.reshape(NW, NCHUNK, CHUNK)

    xj, xi = _sc_gather_xixj(atom_fea, src3, dst3)

    wfi, wfj, wfe = Wf[:, :D].T, Wf[:, D:2 * D].T, Wf[:, 2 * D:].T
    wsi, wsj, wse = Ws[:, :D].T, Ws[:, D:2 * D].T, Ws[:, 2 * D:].T
    msg = _tc_message(xi, xj, edge_fea, wfi, wfj, wfe, bf.reshape(1, D),
                      wsi, wsj, wse, bs.reshape(1, D))

    partials = _sc_scatter_add(msg, dst3)

    pad = jnp.zeros((D, HID_PAD - HID), jnp.float32)
    w1ab = jnp.concatenate(
        [W1[:, :D].T, pad, W1[:, D:2 * D].T, pad,
         jnp.zeros((D, D - 2 * HID_PAD), jnp.float32)], axis=1)
    atom_out, qtab = _tc_node_update(partials, atom_fea, w1ab)

    h = _sc_gather_h(qtab, src, dst).reshape(N_EDGES, HID_PAD)

    epad = jnp.zeros((D_EDGE, HID_PAD - HID), jnp.float32)
    w1e = jnp.concatenate([W1[:, 2 * D:].T, epad], axis=1)
    b1v = jnp.concatenate([b1, jnp.zeros((HID_PAD - HID,), jnp.float32)])
    w2 = jnp.concatenate([W2.T, jnp.zeros((HID_PAD - HID, D), jnp.float32)],
                         axis=0)
    edge_out = _tc_edge_mlp(h, edge_fea, w1e, b1v.reshape(1, HID_PAD),
                            w2, b2.reshape(1, D))
    return atom_out, edge_out


# speedup vs baseline: 1.1660x; 1.0237x over previous
import functools

import jax
import jax.numpy as jnp
from jax import lax
from jax.experimental import pallas as pl
from jax.experimental.pallas import tpu as pltpu
from jax.experimental.pallas import tpu_sc as plsc

N_NODES = 10000
N_EDGES = 320000
D = 128
D_EDGE = 16
HID = 14
HID_PAD = 16

NC = 2
NS = 16
NW = NC * NS
EPW = N_EDGES // NW
CHUNK = 80
NCHUNK = EPW // CHUNK
STRIPE = 624
STRIPE_REM = N_NODES - NS * STRIPE
Z_ROWS = 16

_SC_MESH = dict(core_axis_name="c", subcore_axis_name="s")


_NSET = 4


def _sc_gather_xixj(atom, src3, dst3):
    @functools.partial(
        pl.kernel,
        out_type=[jax.ShapeDtypeStruct((N_EDGES, D), jnp.float32),
                  jax.ShapeDtypeStruct((N_EDGES, D), jnp.float32)],
        mesh=plsc.VectorSubcoreMesh(**_SC_MESH),
        scratch_types=(
            [pltpu.VMEM((NCHUNK, CHUNK), jnp.int32)] * 2
            + [pltpu.VMEM((CHUNK, D), jnp.float32)] * (2 * _NSET)
            + [pltpu.SemaphoreType.DMA] * (2 * _NSET)
        ),
    )
    def k(atom_hbm, src_hbm, dst_hbm, xj_hbm, xi_hbm, idx_s, idx_d, *rest):
        bufs = rest[:2 * _NSET]
        gsems = rest[2 * _NSET:3 * _NSET]
        wsems = rest[3 * _NSET:]
        sets = [(bufs[2 * i], bufs[2 * i + 1], gsems[i], wsems[i])
                for i in range(_NSET)]
        wid = lax.axis_index("s") * NC + lax.axis_index("c")
        base = wid * EPW
        pltpu.sync_copy(src_hbm.at[wid], idx_s)
        pltpu.sync_copy(dst_hbm.at[wid], idx_d)

        def gi(j, S):
            bs, bd, gs, _ = S
            pltpu.async_copy(atom_hbm.at[idx_s.at[j]], bs, gs)
            pltpu.async_copy(atom_hbm.at[idx_d.at[j]], bd, gs)

        def gdrain(S):
            bs, bd, gs, _ = S
            pltpu.make_async_copy(atom_hbm.at[pl.ds(0, CHUNK)], bs, gs).wait()
            pltpu.make_async_copy(atom_hbm.at[pl.ds(0, CHUNK)], bd, gs).wait()

        def wbi(j, S):
            bs, bd, _, ws = S
            off = base + j * CHUNK
            pltpu.async_copy(bs, xj_hbm.at[pl.ds(off, CHUNK)], ws)
            pltpu.async_copy(bd, xi_hbm.at[pl.ds(off, CHUNK)], ws)

        def wdrain(S):
            bs, bd, _, ws = S
            pltpu.make_async_copy(bs, xj_hbm.at[pl.ds(0, CHUNK)], ws).wait()
            pltpu.make_async_copy(bd, xi_hbm.at[pl.ds(0, CHUNK)], ws).wait()

        for kk in range(_NSET):
            gi(kk, sets[kk])

        nbody = (NCHUNK - 1) // _NSET - 1

        def body(t, _):
            j = _NSET * t
            for kk in range(_NSET):
                gdrain(sets[kk])
                wbi(j + kk, sets[kk])
            for kk in range(_NSET):
                wdrain(sets[kk])
                gi(j + _NSET + kk, sets[kk])
            return 0

        lax.fori_loop(0, nbody, body, 0)
        jlast = _NSET * nbody
        for kk in range(_NSET):
            gdrain(sets[kk])
            wbi(jlast + kk, sets[kk])
        wdrain(sets[0])
        gi(NCHUNK - 1, sets[0])
        for kk in range(1, _NSET):
            wdrain(sets[kk])
        gdrain(sets[0])
        wbi(NCHUNK - 1, sets[0])
        wdrain(sets[0])

    return k(atom, src3, dst3)


_MSG_BLK = 4000


def _tc_message(xi, xj, ef, wfi, wfj, wfe, bfv, wsi, wsj, wse, bsv):
    def body(xi_ref, xj_ref, ef_ref, wfi_ref, wfj_ref, wfe_ref, bf_ref,
             wsi_ref, wsj_ref, wse_ref, bs_ref, out_ref):
        bft = jnp.bfloat16
        xi_b = xi_ref[...].astype(bft)
        xj_b = xj_ref[...].astype(bft)
        ef_b = ef_ref[...].astype(bft)
        dot = functools.partial(jnp.dot, preferred_element_type=jnp.float32)
        pf = (dot(xi_b, wfi_ref[...].astype(bft))
              + dot(xj_b, wfj_ref[...].astype(bft))
              + dot(ef_b, wfe_ref[...].astype(bft)) + bf_ref[...])
        ps = (dot(xi_b, wsi_ref[...].astype(bft))
              + dot(xj_b, wsj_ref[...].astype(bft))
              + dot(ef_b, wse_ref[...].astype(bft)) + bs_ref[...])
        sp = jnp.maximum(ps, 0.0) + jnp.log1p(jnp.exp(-jnp.abs(ps)))
        out_ref[...] = jax.nn.sigmoid(pf) * sp

    full = lambda shape: pl.BlockSpec(shape, lambda i: (0, 0))
    return pl.pallas_call(
        body,
        grid=(N_EDGES // _MSG_BLK,),
        in_specs=[
            pl.BlockSpec((_MSG_BLK, D), lambda i: (i, 0)),
            pl.BlockSpec((_MSG_BLK, D), lambda i: (i, 0)),
            pl.BlockSpec((_MSG_BLK, D_EDGE), lambda i: (i, 0)),
            full((D, D)), full((D, D)), full((D_EDGE, D)), full((1, D)),
            full((D, D)), full((D, D)), full((D_EDGE, D)), full((1, D)),
        ],
        out_specs=pl.BlockSpec((_MSG_BLK, D), lambda i: (i, 0)),
        out_shape=jax.ShapeDtypeStruct((N_EDGES, D), jnp.float32),
        compiler_params=pltpu.CompilerParams(
            dimension_semantics=("arbitrary",)),
    )(xi, xj, ef, wfi, wfj, wfe, bfv, wsi, wsj, wse, bsv)


def _sc_scatter_add(msg, dst3):
    @functools.partial(
        pl.kernel,
        out_type=jax.ShapeDtypeStruct((NC, N_NODES, D), jnp.float32),
        mesh=plsc.VectorSubcoreMesh(**_SC_MESH),
        scratch_types=[
            pltpu.VMEM((CHUNK, D), jnp.float32),
            pltpu.VMEM((CHUNK, D), jnp.float32),
            pltpu.VMEM((NCHUNK, CHUNK), jnp.int32),
            pltpu.VMEM((Z_ROWS, D), jnp.float32),
            pltpu.VMEM_SHARED((N_NODES, D), jnp.float32),
            pltpu.SemaphoreType.DMA,
            pltpu.SemaphoreType.DMA,
        ],
    )
    def k(msg_hbm, dst_hbm, out_hbm, m_a, m_b, idxbuf, zbuf, agg,
          sem_a, sem_b):
        c = lax.axis_index("c")
        s = lax.axis_index("s")
        wid = s * NC + c

        zero = jnp.zeros((16,), jnp.float32)

        def zrow(r, _):
            for cc in range(D // 16):
                zbuf[r, pl.ds(cc * 16, 16)] = zero
            return 0

        lax.fori_loop(0, Z_ROWS, zrow, 0)
        for t in range(STRIPE // Z_ROWS):
            pltpu.sync_copy(zbuf, agg.at[pl.ds(s * STRIPE + t * Z_ROWS, Z_ROWS)])

        @pl.when(s == NS - 1)
        def _zero_tail():
            pltpu.sync_copy(zbuf.at[pl.ds(0, STRIPE_REM)],
                            agg.at[pl.ds(NS * STRIPE, STRIPE_REM)])

        plsc.subcore_barrier()

        pltpu.sync_copy(dst_hbm.at[wid], idxbuf)
        base = wid * EPW

        def issue(j, buf, sem):
            pltpu.async_copy(msg_hbm.at[pl.ds(base + j * CHUNK, CHUNK)],
                             buf, sem)

        def drain(buf, sem):
            pltpu.make_async_copy(msg_hbm.at[pl.ds(0, CHUNK)], buf, sem).wait()

        def scat(j, buf):
            pltpu.sync_copy(buf, agg.at[idxbuf.at[j]], add=True)

        issue(0, m_a, sem_a)

        def body(t, _):
            j = 2 * t
            issue(j + 1, m_b, sem_b)
            drain(m_a, sem_a)
            scat(j, m_a)

            @pl.when(j + 2 < NCHUNK)
            def _():
                issue(j + 2, m_a, sem_a)

            drain(m_b, sem_b)
            scat(j + 1, m_b)
            return 0

        lax.fori_loop(0, NCHUNK // 2, body, 0)
        drain(m_a, sem_a)
        scat(NCHUNK - 1, m_a)
        plsc.subcore_barrier()

        pltpu.sync_copy(agg.at[pl.ds(s * STRIPE, STRIPE)],
                        out_hbm.at[c, pl.ds(s * STRIPE, STRIPE)])

        @pl.when(s == NS - 1)
        def _flush_tail():
            pltpu.sync_copy(agg.at[pl.ds(NS * STRIPE, STRIPE_REM)],
                            out_hbm.at[c, pl.ds(NS * STRIPE, STRIPE_REM)])

    return k(msg, dst3)


def _tc_node_update(partials, atom, w1ab):
    def body(p_ref, atom_ref, w1ab_ref, out_ref, q_ref):
        p = p_ref[...]
        ao = p[0] + p[1] + atom_ref[...]
        out_ref[...] = ao
        q_ref[...] = ao @ w1ab_ref[...]

    return pl.pallas_call(
        body,
        out_shape=[jax.ShapeDtypeStruct((N_NODES, D), jnp.float32),
                   jax.ShapeDtypeStruct((N_NODES, D), jnp.float32)],
    )(partials, atom, w1ab)


QCH = 64
NQCH = N_EDGES // QCH
QROWS = QCH * HID_PAD // D
H_ROWS = N_EDGES * HID_PAD // D


def _sc_gather_h(qtab, sd):
    @functools.partial(
        pl.kernel,
        out_type=jax.ShapeDtypeStruct((H_ROWS, D), jnp.float32),
        mesh=plsc.VectorSubcoreMesh(**_SC_MESH),
        scratch_types=[
            pltpu.VMEM((2 * QCH,), jnp.int32),
            pltpu.VMEM((2 * QCH,), jnp.int32),
            pltpu.VMEM((QCH, D), jnp.float32),
            pltpu.VMEM((QCH, D), jnp.float32),
            pltpu.VMEM((QCH, D), jnp.float32),
            pltpu.VMEM((QCH, D), jnp.float32),
            pltpu.VMEM((QROWS, D), jnp.float32),
            pltpu.VMEM((QROWS, D), jnp.float32),
            pltpu.SemaphoreType.DMA,
            pltpu.SemaphoreType.DMA,
            pltpu.SemaphoreType.DMA,
            pltpu.SemaphoreType.DMA,
        ],
    )
    def k(q_hbm, sd_hbm, h_hbm,
          isd_a, isd_b, s_a, d_a, s_b, d_b, h_a, h_b,
          sem_a, sem_b, semw_a, semw_b):
        wid = lax.axis_index("s") * NC + lax.axis_index("c")

        def issue(g, isd, bs, bd, sem):
            pltpu.sync_copy(sd_hbm.at[g], isd)
            pltpu.async_copy(q_hbm.at[isd.at[pl.ds(0, QCH)]], bs, sem)
            pltpu.async_copy(q_hbm.at[isd.at[pl.ds(QCH, QCH)]], bd, sem)

        def drain(bs, bd, sem):
            pltpu.make_async_copy(q_hbm.at[pl.ds(0, QCH)], bs, sem).wait()
            pltpu.make_async_copy(q_hbm.at[pl.ds(0, QCH)], bd, sem).wait()

        def addwb(g, bs, bd, hb, semw):
            for e in range(QCH):
                hb[e // 8, pl.ds((e % 8) * HID_PAD, HID_PAD)] = (
                    bs[e, pl.ds(0, HID_PAD)] + bd[e, pl.ds(HID_PAD, HID_PAD)])
            pltpu.async_copy(hb, h_hbm.at[pl.ds(g * QROWS, QROWS)], semw)

        def wdrain(hb, semw):
            pltpu.make_async_copy(hb, h_hbm.at[pl.ds(0, QROWS)], semw).wait()

        issue(wid, isd_a, s_a, d_a, sem_a)

        def body(t, _):
            g0 = (2 * t) * NW + wid
            g1 = g0 + NW
            g2 = g1 + NW
            issue(g1, isd_b, s_b, d_b, sem_b)
            drain(s_a, d_a, sem_a)

            @pl.when(t > 0)
            def _():
                wdrain(h_a, semw_a)

            addwb(g0, s_a, d_a, h_a, semw_a)

            @pl.when(g2 < NQCH)
            def _():
                issue(g2, isd_a, s_a, d_a, sem_a)

            drain(s_b, d_b, sem_b)

            @pl.when(t > 0)
            def _():
                wdrain(h_b, semw_b)

            addwb(g1, s_b, d_b, h_b, semw_b)
            return 0

        lax.fori_loop(0, 78, body, 0)
        wdrain(h_b, semw_b)
        glast = 156 * NW + wid

        @pl.when(glast < NQCH)
        def _tail():
            drain(s_a, d_a, sem_a)
            wdrain(h_a, semw_a)
            addwb(glast, s_a, d_a, h_a, semw_a)

        wdrain(h_a, semw_a)

    return k(qtab, sd)


_EDGE_BLK = 4000


def _tc_edge_mlp(h, ef, w1e, b1v, w2, b2v):
    def body(h_ref, ef_ref, w1e_ref, b1_ref, w2_ref, b2_ref, out_ref):
        hp = h_ref[...] + ef_ref[...] @ w1e_ref[...] + b1_ref[...]
        hh = hp * jax.nn.sigmoid(hp)
        o = hh @ w2_ref[...] + b2_ref[...]
        out_ref[...] = o * jax.nn.sigmoid(o)

    full = lambda shape: pl.BlockSpec(shape, lambda i: (0, 0))
    return pl.pallas_call(
        body,
        grid=(N_EDGES // _EDGE_BLK,),
        in_specs=[
            pl.BlockSpec((_EDGE_BLK, HID_PAD), lambda i: (i, 0)),
            pl.BlockSpec((_EDGE_BLK, D_EDGE), lambda i: (i, 0)),
            full((D_EDGE, HID_PAD)), full((1, HID_PAD)),
            full((HID_PAD, D)), full((1, D)),
        ],
        out_specs=pl.BlockSpec((_EDGE_BLK, D), lambda i: (i, 0)),
        out_shape=jax.ShapeDtypeStruct((N_EDGES, D), jnp.float32),
        compiler_params=pltpu.CompilerParams(
            dimension_semantics=("arbitrary",)),
    )(h, ef, w1e, b1v, w2, b2v)


def kernel(atom_fea, edge_idx, edge_fea, batch, distance, edge_vec,
           Wf, bf, Ws, bs, W1, b1, W2, b2):
    src = edge_idx[0].astype(jnp.int32)
    dst = edge_idx[1].astype(jnp.int32)
    src3 = src.reshape(NW, NCHUNK, CHUNK)
    dst3 = dst.reshape(NW, NCHUNK, CHUNK)

    xj, xi = _sc_gather_xixj(atom_fea, src3, dst3)

    wfi, wfj, wfe = Wf[:, :D].T, Wf[:, D:2 * D].T, Wf[:, 2 * D:].T
    wsi, wsj, wse = Ws[:, :D].T, Ws[:, D:2 * D].T, Ws[:, 2 * D:].T
    msg = _tc_message(xi, xj, edge_fea, wfi, wfj, wfe, bf.reshape(1, D),
                      wsi, wsj, wse, bs.reshape(1, D))

    partials = _sc_scatter_add(msg, dst3)

    pad = jnp.zeros((D, HID_PAD - HID), jnp.float32)
    w1ab = jnp.concatenate(
        [W1[:, :D].T, pad, W1[:, D:2 * D].T, pad,
         jnp.zeros((D, D - 2 * HID_PAD), jnp.float32)], axis=1)
    atom_out, qtab = _tc_node_update(partials, atom_fea, w1ab)

    sd = jnp.concatenate([src.reshape(NQCH, QCH), dst.reshape(NQCH, QCH)],
                         axis=1)
    h = _sc_gather_h(qtab, sd).reshape(N_EDGES, HID_PAD)

    epad = jnp.zeros((D_EDGE, HID_PAD - HID), jnp.float32)
    w1e = jnp.concatenate([W1[:, 2 * D:].T, epad], axis=1)
    b1v = jnp.concatenate([b1, jnp.zeros((HID_PAD - HID,), jnp.float32)])
    w2 = jnp.concatenate([W2.T, jnp.zeros((HID_PAD - HID, D), jnp.float32)],
                         axis=0)
    edge_out = _tc_edge_mlp(h, edge_fea, w1e, b1v.reshape(1, HID_PAD),
                            w2, b2.reshape(1, D))
    return atom_out, edge_out
